# Initial kernel scaffold; baseline (speedup 1.0000x reference)
import jax
import jax.numpy as jnp
from jax.experimental import pallas as pl


def kernel(Lap, x, W1, b1, W2, b2, tg, Wg, bg, tl, Wl, bl, Wp1, bp1, Wp2, bp2):
    raise NotImplementedError("write your pallas kernel here")


# speedup vs baseline: 1.2786x; 1.2786x over previous
import jax
import jax.numpy as jnp
from jax.experimental import pallas as pl
from jax.experimental.pallas import tpu as pltpu

_N = 4096
_TAU = 0.2
_BE = 512
_BM = 256
_BL = 512
_F32 = jnp.float32


def _relu(v):
    return jnp.maximum(v, 0.0)


def _enc_body(x_ref, w1_ref, b1_ref, w2_ref, b2_ref, h_ref):
    h = _relu(jnp.dot(x_ref[...], w1_ref[...], preferred_element_type=_F32)
              + b1_ref[...])
    h = jnp.dot(h, w2_ref[...], preferred_element_type=_F32) + b2_ref[...]
    h_ref[...] = _relu(h)


def _diff1_body(lap_ref, h_ref, wg_ref, bg_ref, tg_ref, wl_ref, bl_ref,
                tl_ref, xg_ref, xl_ref):
    m = pl.program_id(0)
    lx = jnp.dot(lap_ref[...], h_ref[...], preferred_element_type=_F32)
    hm = h_ref[pl.ds(m * _BM, _BM), :]
    ug = hm - tg_ref[...] * lx
    ul = hm - tl_ref[...] * lx
    xg_ref[...] = _relu(
        jnp.dot(ug, wg_ref[...], preferred_element_type=_F32) + bg_ref[...])
    xl_ref[...] = _relu(
        jnp.dot(ul, wl_ref[...], preferred_element_type=_F32) + bl_ref[...])


def _diff2_body(lap_ref, xgf_ref, xlf_ref, wg_ref, bg_ref, tg_ref, wl_ref,
                bl_ref, tl_ref, wp1_ref, bp1_ref, wp2_ref, bp2_ref,
                xg_ref, xl_ref, g_ref, l_ref):
    m = pl.program_id(0)
    a = lap_ref[...]
    lxg = jnp.dot(a, xgf_ref[...], preferred_element_type=_F32)
    lxl = jnp.dot(a, xlf_ref[...], preferred_element_type=_F32)
    xgm = xgf_ref[pl.ds(m * _BM, _BM), :]
    xlm = xlf_ref[pl.ds(m * _BM, _BM), :]
    ug = xgm - tg_ref[...] * lxg
    ul = xlm - tl_ref[...] * lxl
    xg = _relu(
        jnp.dot(ug, wg_ref[...], preferred_element_type=_F32) + bg_ref[...])
    xl = _relu(
        jnp.dot(ul, wl_ref[...], preferred_element_type=_F32) + bl_ref[...])
    xg_ref[...] = xg
    xl_ref[...] = xl
    zg = jnp.dot(xg, wp1_ref[...], preferred_element_type=_F32) + bp1_ref[...]
    zl = jnp.dot(xl, wp2_ref[...], preferred_element_type=_F32) + bp2_ref[...]
    ng = jnp.sqrt(jnp.sum(zg * zg, axis=1, keepdims=True))
    nl = jnp.sqrt(jnp.sum(zl * zl, axis=1, keepdims=True))
    g_ref[...] = zg / jnp.maximum(ng, 1e-12)
    l_ref[...] = zl / jnp.maximum(nl, 1e-12)


def _loss_body(g_ref, l_ref, loss_ref, row_a, row_b, row_c, col_c, diag_s):
    i = pl.program_id(0)
    j = pl.program_id(1)
    gi = g_ref[pl.ds(i * _BL, _BL), :]
    gj = g_ref[pl.ds(j * _BL, _BL), :]
    li = l_ref[pl.ds(i * _BL, _BL), :]
    lj = l_ref[pl.ds(j * _BL, _BL), :]
    dn = (((1,), (1,)), ((), ()))
    sim_a = jax.lax.dot_general(gi, gj, dn, preferred_element_type=_F32) / _TAU
    sim_b = jax.lax.dot_general(li, lj, dn, preferred_element_type=_F32) / _TAU
    sim_c = jax.lax.dot_general(gi, lj, dn, preferred_element_type=_F32) / _TAU
    rows = jax.lax.broadcasted_iota(jnp.int32, (_BL, _BL), 0) + i * _BL
    cols = jax.lax.broadcasted_iota(jnp.int32, (_BL, _BL), 1) + j * _BL
    off = rows != cols
    ea = jnp.where(off, jnp.exp(sim_a), 0.0)
    eb = jnp.where(off, jnp.exp(sim_b), 0.0)
    ec = jnp.where(off, jnp.exp(sim_c), 0.0)
    sa = jnp.sum(ea, axis=1).reshape(1, _BL)
    sb = jnp.sum(eb, axis=1).reshape(1, _BL)
    sc = jnp.sum(ec, axis=1).reshape(1, _BL)
    cc = jnp.sum(ec, axis=0).reshape(1, _BL)

    @pl.when(j == 0)
    def _init_rows():
        row_a[pl.ds(i, 1), :] = sa
        row_b[pl.ds(i, 1), :] = sb
        row_c[pl.ds(i, 1), :] = sc

    @pl.when(j > 0)
    def _acc_rows():
        row_a[pl.ds(i, 1), :] += sa
        row_b[pl.ds(i, 1), :] += sb
        row_c[pl.ds(i, 1), :] += sc

    @pl.when(i == 0)
    def _init_cols():
        col_c[pl.ds(j, 1), :] = cc

    @pl.when(i > 0)
    def _acc_cols():
        col_c[pl.ds(j, 1), :] += cc

    @pl.when(jnp.logical_and(i == 0, j == 0))
    def _init_diag():
        diag_s[0, 0] = 0.0

    @pl.when(i == j)
    def _acc_diag():
        diag_s[0, 0] += jnp.sum(jnp.where(off, 0.0, sim_c))

    nb = _N // _BL

    @pl.when(jnp.logical_and(i == nb - 1, j == nb - 1))
    def _finalize():
        tot = (2.0 * diag_s[0, 0]
               - jnp.sum(jnp.log(row_c[...]))
               - jnp.sum(jnp.log(row_a[...]))
               - jnp.sum(jnp.log(col_c[...]))
               - jnp.sum(jnp.log(row_b[...])))
        loss_ref[...] = jnp.reshape(-0.5 * tot / _N, (1, 1))


def kernel(Lap, x, W1, b1, W2, b2, tg, Wg, bg, tl, Wl, bl, Wp1, bp1, Wp2,
           bp2):
    n, d_in = x.shape
    h1 = W1.shape[0]
    h2 = W2.shape[0]

    full = lambda shape: pl.BlockSpec(shape, lambda *_: (0,) * len(shape))

    h = pl.pallas_call(
        _enc_body,
        grid=(n // _BE,),
        in_specs=[
            pl.BlockSpec((_BE, d_in), lambda m: (m, 0)),
            full((d_in, h1)),
            full((1, h1)),
            full((h1, h2)),
            full((1, h2)),
        ],
        out_specs=pl.BlockSpec((_BE, h2), lambda m: (m, 0)),
        out_shape=jax.ShapeDtypeStruct((n, h2), _F32),
    )(x, W1.T, b1.reshape(1, h1), W2.T, b2.reshape(1, h2))

    tgv = [jnp.broadcast_to(tg[i], (1, h2)).astype(_F32) for i in range(2)]
    tlv = [tl[i].reshape(1, h2) for i in range(2)]

    xg1, xl1 = pl.pallas_call(
        _diff1_body,
        grid=(n // _BM,),
        in_specs=[
            pl.BlockSpec((_BM, n), lambda m: (m, 0)),
            full((n, h2)),
            full((h2, h2)), full((1, h2)), full((1, h2)),
            full((h2, h2)), full((1, h2)), full((1, h2)),
        ],
        out_specs=[pl.BlockSpec((_BM, h2), lambda m: (m, 0))] * 2,
        out_shape=[jax.ShapeDtypeStruct((n, h2), _F32)] * 2,
    )(Lap, h, Wg[0].T, bg[0].reshape(1, h2), tgv[0],
      Wl[0].T, bl[0].reshape(1, h2), tlv[0])

    xg2, xl2, g, l = pl.pallas_call(
        _diff2_body,
        grid=(n // _BM,),
        in_specs=[
            pl.BlockSpec((_BM, n), lambda m: (m, 0)),
            full((n, h2)),
            full((n, h2)),
            full((h2, h2)), full((1, h2)), full((1, h2)),
            full((h2, h2)), full((1, h2)), full((1, h2)),
            full((h2, h2)), full((1, h2)),
            full((h2, h2)), full((1, h2)),
        ],
        out_specs=[pl.BlockSpec((_BM, h2), lambda m: (m, 0))] * 4,
        out_shape=[jax.ShapeDtypeStruct((n, h2), _F32)] * 4,
    )(Lap, xg1, xl1, Wg[1].T, bg[1].reshape(1, h2), tgv[1],
      Wl[1].T, bl[1].reshape(1, h2), tlv[1],
      Wp1.T, bp1.reshape(1, h2), Wp2.T, bp2.reshape(1, h2))

    nb = n // _BL
    loss2d = pl.pallas_call(
        _loss_body,
        grid=(nb, nb),
        in_specs=[full((n, h2)), full((n, h2))],
        out_specs=pl.BlockSpec((1, 1), lambda i, j: (0, 0)),
        out_shape=jax.ShapeDtypeStruct((1, 1), _F32),
        scratch_shapes=[
            pltpu.VMEM((nb, _BL), _F32),
            pltpu.VMEM((nb, _BL), _F32),
            pltpu.VMEM((nb, _BL), _F32),
            pltpu.VMEM((nb, _BL), _F32),
            pltpu.SMEM((1, 1), _F32),
        ],
    )(g, l)

    return (xg2, xl2, loss2d[0, 0])
